# baseline (speedup 1.0000x reference)
import numpy as np

import jax
import jax.numpy as jnp
from jax import lax
from jax.experimental import pallas as pl
from jax.experimental.pallas import tpu as pltpu
from jax.experimental.pallas import tpu_sc as plsc

N_USERS = 5000
N_NODES = 10000
N_EDGES = 160000
EMB = 256
BLK = 64
N_BLK = EMB // BLK
N_LAYERS = 3

NS = 16
N_PAD = 10240
ROWS_PER_TILE = N_PAD // NS
EDGES_PER_TILE = N_EDGES // NS
CHUNK = 80
N_CHUNKS = EDGES_PER_TILE // CHUNK
WB = 32
N_WB = ROWS_PER_TILE // WB
NV = BLK // 16
NBUF = 3
ZB = 16
T_TRIPLES = (N_CHUNKS - 2) // NBUF
L_TAIL = T_TRIPLES * NBUF

_SM = np.empty((EMB,), np.int32)
_CM = np.empty((EMB,), np.int32)
for _g in range(EMB // 32):
    for _k in range(16):
        _SM[32 * _g + _k] = 32 * _g + 2 * _k
        _SM[32 * _g + 16 + _k] = 32 * _g + 2 * _k + 1
        _CM[32 * _g + 2 * _k] = 32 * _g + _k
        _CM[32 * _g + 2 * _k + 1] = 32 * _g + 16 + _k


def _lightgcn_body(tb0, tb1, tb2, tb3, sd0, sd1, sd2, sd3, src, dst, w,
                   out0, out1, out2, out3, eb0, eb1, eb2, eb3,
                   accum, sumv, srcall, dstall,
                   gb0, gb1, gb2, sb0, sb1, sb2, wb0, wb1, wb2,
                   tmp, tmpbf, zbuf,
                   gsem0, gsem1, gsem2, ssem0, ssem1, ssem2):
    c = lax.axis_index("c")
    s = lax.axis_index("s")
    gbufs = [gb0, gb1, gb2]
    sbufs = [sb0, sb1, sb2]
    wbufs = [wb0, wb1, wb2]
    gsem = [gsem0, gsem1, gsem2]
    ssem = [ssem0, ssem1, ssem2]

    r0 = s * ROWS_PER_TILE
    i0 = s * N_CHUNKS

    pltpu.sync_copy(src.at[pl.ds(i0, N_CHUNKS)], srcall)
    pltpu.sync_copy(dst.at[pl.ds(i0, N_CHUNKS)], dstall)

    def zrow(i, _):
        for q in range(NV):
            zbuf[i, pl.ds(q * 16, 16)] = jnp.zeros((16,), jnp.float32)
        return 0
    lax.fori_loop(0, ZB, zrow, 0)

    def run(tbl, seed, ebuf, out):
        def gather_issue(i, b):
            pltpu.async_copy(ebuf.at[srcall.at[i]], gbufs[b], gsem[b])
            pltpu.async_copy(w.at[i0 + i], wbufs[b], gsem[b])

        def gather_wait(i, b):
            pltpu.make_async_copy(ebuf.at[srcall.at[i]], gbufs[b],
                                  gsem[b]).wait()
            pltpu.make_async_copy(w.at[i0 + i], wbufs[b], gsem[b]).wait()

        def scatter_issue(i, b):
            pltpu.async_copy(sbufs[b], accum.at[dstall.at[i]], ssem[b],
                             add=True)

        def scatter_wait(i, b):
            pltpu.make_async_copy(sbufs[b], accum.at[dstall.at[i]],
                                  ssem[b]).wait()

        def scale(i, b):
            gb, sb = gbufs[b], sbufs[b]

            wvb = wbufs[b]

            def body(j, _):
                wvec = plsc.load_gather(wvb, [jnp.broadcast_to(j, (16,))])
                for g in range(BLK // 32):
                    x = gb[j, pl.ds(32 * g, 32)]
                    va, vb = plsc.unpack(x, format=plsc.PackFormat.INTERLEAVED)
                    sb[j, pl.ds(32 * g, 16)] = va * wvec
                    sb[j, pl.ds(32 * g + 16, 16)] = vb * wvec
                return 0
            lax.fori_loop(0, CHUNK, body, 0)

        pltpu.sync_copy(seed.at[pl.ds(r0, ROWS_PER_TILE)], sumv)
        for k in range(ROWS_PER_TILE // CHUNK):
            pltpu.sync_copy(tbl.at[pl.ds(r0 + k * CHUNK, CHUNK)], gb0)
            pltpu.sync_copy(gb0, ebuf.at[pl.ds(r0 + k * CHUNK, CHUNK)])
        plsc.subcore_barrier()

        def layer_body(_l, _c):
            def zero(b, _):
                pltpu.sync_copy(zbuf, accum.at[pl.ds(r0 + b * ZB, ZB)])
                return 0
            lax.fori_loop(0, ROWS_PER_TILE // ZB, zero, 0)
            plsc.subcore_barrier()

            gather_issue(0, 0)
            gather_issue(1, 1)

            def triple(t, _):
                for slot in range(NBUF):
                    i = NBUF * t + slot
                    gather_issue(i + 2, (slot + 2) % NBUF)
                    gather_wait(i, slot)
                    pl.when(t > 0)(lambda: scatter_wait(i - NBUF, slot))
                    scale(i, slot)
                    scatter_issue(i, slot)
                return 0
            lax.fori_loop(0, T_TRIPLES, triple, 0)

            for i in range(L_TAIL, N_CHUNKS):
                slot = i % NBUF
                if i + 2 < N_CHUNKS:
                    gather_issue(i + 2, (i + 2) % NBUF)
                gather_wait(i, slot)
                scatter_wait(i - NBUF, slot)
                scale(i, slot)
                scatter_issue(i, slot)
            for k in range(N_CHUNKS - NBUF, N_CHUNKS):
                scatter_wait(k, k % NBUF)
            plsc.subcore_barrier()

            def wb(b, _):
                rb = r0 + b * WB
                pltpu.sync_copy(accum.at[pl.ds(rb, WB)], tmp)

                def rep(i, _):
                    for g in range(BLK // 32):
                        va = tmp[i, pl.ds(32 * g, 16)]
                        vb = tmp[i, pl.ds(32 * g + 16, 16)]
                        tmpbf[i, pl.ds(32 * g, 32)] = plsc.pack(
                            va, vb, format=plsc.PackFormat.INTERLEAVED)
                        sl0 = pl.ds(32 * g, 16)
                        sl1 = pl.ds(32 * g + 16, 16)
                        sumv[b * WB + i, sl0] = sumv[b * WB + i, sl0] + va
                        sumv[b * WB + i, sl1] = sumv[b * WB + i, sl1] + vb
                    return 0
                lax.fori_loop(0, WB, rep, 0)
                pltpu.sync_copy(tmpbf, ebuf.at[pl.ds(rb, WB)])
                return 0
            lax.fori_loop(0, N_WB, wb, 0)
            plsc.subcore_barrier()
            return 0
        lax.fori_loop(0, N_LAYERS, layer_body, 0)

        inv = jnp.float32(1.0 / (N_LAYERS + 1))

        def finb(b, _):
            rb = r0 + b * WB

            def fin(i, _):
                for q in range(NV):
                    sl = pl.ds(q * 16, 16)
                    tmp[i, sl] = sumv[b * WB + i, sl] * inv
                return 0
            lax.fori_loop(0, WB, fin, 0)
            pltpu.sync_copy(tmp, out.at[pl.ds(rb, WB)])
            return 0
        lax.fori_loop(0, N_WB, finb, 0)

    def core0():
        run(tb0, sd0, eb0, out0)
        run(tb1, sd1, eb1, out1)

    def core1():
        run(tb2, sd2, eb2, out2)
        run(tb3, sd3, eb3, out3)

    pl.when(c == 0)(core0)
    pl.when(c == 1)(core1)


@jax.jit
def kernel(user_emb, item_emb, edge_src, edge_dst, edge_weight):
    ego = jnp.concatenate([user_emb, item_emb], axis=0)
    ego = jnp.pad(ego, ((0, N_PAD - N_NODES), (0, 0)))
    tables = [ego[:, b * BLK:(b + 1) * BLK].astype(jnp.bfloat16)
              for b in range(N_BLK)]
    ego_sm = ego[:, _SM]
    seeds = [ego_sm[:, b * BLK:(b + 1) * BLK] for b in range(N_BLK)]
    src = edge_src.astype(jnp.int32).reshape(N_EDGES // CHUNK, CHUNK)
    dst = edge_dst.astype(jnp.int32).reshape(N_EDGES // CHUNK, CHUNK)
    w = edge_weight.astype(jnp.float32).reshape(N_EDGES // CHUNK, CHUNK)

    mesh = plsc.VectorSubcoreMesh(core_axis_name="c", subcore_axis_name="s")
    f32 = jnp.float32
    i32 = jnp.int32
    bf16 = jnp.bfloat16
    out_t = jax.ShapeDtypeStruct((N_PAD, BLK), f32)
    eb_t = jax.ShapeDtypeStruct((N_PAD, BLK), bf16)
    call = pl.kernel(
        _lightgcn_body,
        out_type=[out_t] * N_BLK + [eb_t] * N_BLK,
        mesh=mesh,
        compiler_params=pltpu.CompilerParams(
            needs_layout_passes=False, use_tc_tiling_on_sc=False),
        scratch_types=[
            pltpu.VMEM_SHARED((N_PAD, BLK), f32),
            pltpu.VMEM((ROWS_PER_TILE, BLK), f32),
            pltpu.VMEM((N_CHUNKS, CHUNK), i32),
            pltpu.VMEM((N_CHUNKS, CHUNK), i32),
            pltpu.VMEM((CHUNK, BLK), bf16),
            pltpu.VMEM((CHUNK, BLK), bf16),
            pltpu.VMEM((CHUNK, BLK), bf16),
            pltpu.VMEM((CHUNK, BLK), f32),
            pltpu.VMEM((CHUNK, BLK), f32),
            pltpu.VMEM((CHUNK, BLK), f32),
            pltpu.VMEM((CHUNK,), f32),
            pltpu.VMEM((CHUNK,), f32),
            pltpu.VMEM((CHUNK,), f32),
            pltpu.VMEM((WB, BLK), f32),
            pltpu.VMEM((WB, BLK), bf16),
            pltpu.VMEM((ZB, BLK), f32),
            pltpu.SemaphoreType.DMA,
            pltpu.SemaphoreType.DMA,
            pltpu.SemaphoreType.DMA,
            pltpu.SemaphoreType.DMA,
            pltpu.SemaphoreType.DMA,
            pltpu.SemaphoreType.DMA,
        ],
    )
    outs = call(*tables, *seeds, src, dst, w)
    raw = jnp.concatenate(outs[:N_BLK], axis=1)
    mean_emb = raw[:, _CM]
    return (mean_emb[:N_USERS], mean_emb[N_USERS:N_NODES])


# speedup vs baseline: 1.5994x; 1.5994x over previous
import jax
import jax.numpy as jnp
from jax import lax
from jax.experimental import pallas as pl
from jax.experimental.pallas import tpu as pltpu
from jax.experimental.pallas import tpu_sc as plsc

N_USERS = 5000
N_NODES = 10000
N_EDGES = 160000
EMB = 256
BLK = 64
N_BLK = EMB // BLK
N_LAYERS = 3

NS = 16
N_PAD = 10240
ROWS_PER_TILE = N_PAD // NS
EDGES_PER_TILE = N_EDGES // NS
CHUNK = 80
N_CHUNKS = EDGES_PER_TILE // CHUNK
WB = 32
N_WB = ROWS_PER_TILE // WB
NV = BLK // 16
NBUF = 4
ZB = 16
T_RING = (N_CHUNKS - 2) // NBUF
L_TAIL = T_RING * NBUF


def _lightgcn_body(t0, t1, t2, t3, src, dst, w,
                   out0, out1, out2, out3, eb0, eb1, eb2, eb3,
                   accum, sumv, srcall, dstall,
                   rows0, rows1, rows2, rows3, wb0, wb1, wb2, wb3,
                   tmp, zbuf,
                   gsem0, gsem1, gsem2, gsem3, ssem0, ssem1, ssem2, ssem3):
    c = lax.axis_index("c")
    s = lax.axis_index("s")
    rows = [rows0, rows1, rows2, rows3]
    wbufs = [wb0, wb1, wb2, wb3]
    gsem = [gsem0, gsem1, gsem2, gsem3]
    ssem = [ssem0, ssem1, ssem2, ssem3]

    r0 = s * ROWS_PER_TILE
    i0 = s * N_CHUNKS

    pltpu.sync_copy(src.at[pl.ds(i0, N_CHUNKS)], srcall)
    pltpu.sync_copy(dst.at[pl.ds(i0, N_CHUNKS)], dstall)

    def zrow(i, _):
        for q in range(NV):
            zbuf[i, pl.ds(q * 16, 16)] = jnp.zeros((16,), jnp.float32)
        return 0
    lax.fori_loop(0, ZB, zrow, 0)

    def run(tbl, ebuf, out):
        def gather_issue(i, b):
            pltpu.async_copy(ebuf.at[srcall.at[i]], rows[b], gsem[b])
            pltpu.async_copy(w.at[i0 + i], wbufs[b], gsem[b])

        def gather_wait(i, b):
            pltpu.make_async_copy(ebuf.at[srcall.at[i]], rows[b],
                                  gsem[b]).wait()
            pltpu.make_async_copy(w.at[i0 + i], wbufs[b], gsem[b]).wait()

        def scatter_issue(i, b):
            pltpu.async_copy(rows[b], accum.at[dstall.at[i]], ssem[b],
                             add=True)

        def scatter_wait(i, b):
            pltpu.make_async_copy(rows[b], accum.at[dstall.at[i]],
                                  ssem[b]).wait()

        def scale(i, b):
            rb, wvb = rows[b], wbufs[b]

            def body(j, _):
                wvec = plsc.load_gather(wvb, [jnp.broadcast_to(j, (16,))])
                for q in range(NV):
                    sl = pl.ds(q * 16, 16)
                    rb[j, sl] = rb[j, sl] * wvec
                return 0
            lax.fori_loop(0, CHUNK, body, 0)

        pltpu.sync_copy(tbl.at[pl.ds(r0, ROWS_PER_TILE)], sumv)
        pltpu.sync_copy(sumv, ebuf.at[pl.ds(r0, ROWS_PER_TILE)])
        plsc.subcore_barrier()

        def layer_body(_l, _c):
            def zero(b, _):
                pltpu.sync_copy(zbuf, accum.at[pl.ds(r0 + b * ZB, ZB)])
                return 0
            lax.fori_loop(0, ROWS_PER_TILE // ZB, zero, 0)
            plsc.subcore_barrier()

            gather_issue(0, 0)
            gather_issue(1, 1)

            def ring(t, _):
                for slot in range(NBUF):
                    i = NBUF * t + slot
                    nxt = (slot + 2) % NBUF
                    if slot < 2:
                        pl.when(t > 0)(lambda: scatter_wait(i - 2, nxt))
                    else:
                        scatter_wait(i - 2, nxt)
                    gather_issue(i + 2, nxt)
                    gather_wait(i, slot)
                    scale(i, slot)
                    scatter_issue(i, slot)
                return 0
            lax.fori_loop(0, T_RING, ring, 0)

            for i in range(L_TAIL, N_CHUNKS):
                slot = i % NBUF
                if i + 2 < N_CHUNKS:
                    scatter_wait(i - 2, (i + 2) % NBUF)
                    gather_issue(i + 2, (i + 2) % NBUF)
                gather_wait(i, slot)
                scale(i, slot)
                scatter_issue(i, slot)
            for k in range(N_CHUNKS - NBUF, N_CHUNKS):
                scatter_wait(k, k % NBUF)
            plsc.subcore_barrier()

            def wb(b, _):
                rb = r0 + b * WB
                pltpu.sync_copy(accum.at[pl.ds(rb, WB)], tmp)
                pltpu.sync_copy(tmp, ebuf.at[pl.ds(rb, WB)])

                def acc(i, _):
                    for q in range(NV):
                        sl = pl.ds(q * 16, 16)
                        sumv[b * WB + i, sl] = sumv[b * WB + i, sl] + tmp[i, sl]
                    return 0
                lax.fori_loop(0, WB, acc, 0)
                return 0
            lax.fori_loop(0, N_WB, wb, 0)
            plsc.subcore_barrier()
            return 0
        lax.fori_loop(0, N_LAYERS, layer_body, 0)

        inv = jnp.float32(1.0 / (N_LAYERS + 1))

        def finb(b, _):
            rb = r0 + b * WB

            def fin(i, _):
                for q in range(NV):
                    sl = pl.ds(q * 16, 16)
                    tmp[i, sl] = sumv[b * WB + i, sl] * inv
                return 0
            lax.fori_loop(0, WB, fin, 0)
            pltpu.sync_copy(tmp, out.at[pl.ds(rb, WB)])
            return 0
        lax.fori_loop(0, N_WB, finb, 0)

    def core0():
        run(t0, eb0, out0)
        run(t1, eb1, out1)

    def core1():
        run(t2, eb2, out2)
        run(t3, eb3, out3)

    pl.when(c == 0)(core0)
    pl.when(c == 1)(core1)


@jax.jit
def kernel(user_emb, item_emb, edge_src, edge_dst, edge_weight):
    ego = jnp.concatenate([user_emb, item_emb], axis=0)
    ego = jnp.pad(ego, ((0, N_PAD - N_NODES), (0, 0)))
    tables = [ego[:, b * BLK:(b + 1) * BLK] for b in range(N_BLK)]
    src = edge_src.astype(jnp.int32).reshape(N_EDGES // CHUNK, CHUNK)
    dst = edge_dst.astype(jnp.int32).reshape(N_EDGES // CHUNK, CHUNK)
    w = edge_weight.astype(jnp.float32).reshape(N_EDGES // CHUNK, CHUNK)

    mesh = plsc.VectorSubcoreMesh(core_axis_name="c", subcore_axis_name="s")
    f32 = jnp.float32
    i32 = jnp.int32
    blk_t = jax.ShapeDtypeStruct((N_PAD, BLK), f32)
    call = pl.kernel(
        _lightgcn_body,
        out_type=[blk_t] * 8,
        mesh=mesh,
        compiler_params=pltpu.CompilerParams(
            needs_layout_passes=False, use_tc_tiling_on_sc=False),
        scratch_types=[
            pltpu.VMEM_SHARED((N_PAD, BLK), f32),
            pltpu.VMEM((ROWS_PER_TILE, BLK), f32),
            pltpu.VMEM((N_CHUNKS, CHUNK), i32),
            pltpu.VMEM((N_CHUNKS, CHUNK), i32),
            pltpu.VMEM((CHUNK, BLK), f32),
            pltpu.VMEM((CHUNK, BLK), f32),
            pltpu.VMEM((CHUNK, BLK), f32),
            pltpu.VMEM((CHUNK, BLK), f32),
            pltpu.VMEM((CHUNK,), f32),
            pltpu.VMEM((CHUNK,), f32),
            pltpu.VMEM((CHUNK,), f32),
            pltpu.VMEM((CHUNK,), f32),
            pltpu.VMEM((WB, BLK), f32),
            pltpu.VMEM((ZB, BLK), f32),
            pltpu.SemaphoreType.DMA,
            pltpu.SemaphoreType.DMA,
            pltpu.SemaphoreType.DMA,
            pltpu.SemaphoreType.DMA,
            pltpu.SemaphoreType.DMA,
            pltpu.SemaphoreType.DMA,
            pltpu.SemaphoreType.DMA,
            pltpu.SemaphoreType.DMA,
        ],
    )
    outs = call(*tables, src, dst, w)
    mean_emb = jnp.concatenate(outs[:N_BLK], axis=1)
    return (mean_emb[:N_USERS], mean_emb[N_USERS:N_NODES])


# speedup vs baseline: 1.7619x; 1.1016x over previous
import jax
import jax.numpy as jnp
from jax import lax
from jax.experimental import pallas as pl
from jax.experimental.pallas import tpu as pltpu
from jax.experimental.pallas import tpu_sc as plsc

N_USERS = 5000
N_NODES = 10000
N_EDGES = 160000
EMB = 256
BLK = 64
N_BLK = EMB // BLK
N_LAYERS = 3

NS = 16
N_PAD = 10240
ROWS_PER_TILE = N_PAD // NS
EDGES_PER_TILE = N_EDGES // NS
CHUNK = 80
N_CHUNKS = EDGES_PER_TILE // CHUNK
WB = 64
N_WB = ROWS_PER_TILE // WB
NV = BLK // 16
NBUF = 4
ZB = 64
T_RING = (N_CHUNKS - 2) // NBUF
L_TAIL = T_RING * NBUF


def _lightgcn_body(t0, t1, t2, t3, src, dst, w,
                   out0, out1, out2, out3, eb0, eb1, eb2, eb3,
                   accum, sumv, srcall, dstall,
                   rows0, rows1, rows2, rows3, wb0, wb1, wb2, wb3,
                   tmp, zbuf,
                   gsem0, gsem1, gsem2, gsem3, ssem0, ssem1, ssem2, ssem3):
    c = lax.axis_index("c")
    s = lax.axis_index("s")
    rows = [rows0, rows1, rows2, rows3]
    wbufs = [wb0, wb1, wb2, wb3]
    gsem = [gsem0, gsem1, gsem2, gsem3]
    ssem = [ssem0, ssem1, ssem2, ssem3]

    r0 = s * ROWS_PER_TILE
    i0 = s * N_CHUNKS

    pltpu.sync_copy(src.at[pl.ds(i0, N_CHUNKS)], srcall)
    pltpu.sync_copy(dst.at[pl.ds(i0, N_CHUNKS)], dstall)

    def zrow(i, _):
        for q in range(NV):
            zbuf[i, pl.ds(q * 16, 16)] = jnp.zeros((16,), jnp.float32)
        return 0
    lax.fori_loop(0, ZB, zrow, 0)

    def run(tbl, ebuf, out):
        def gather_issue(i, b):
            pltpu.async_copy(ebuf.at[srcall.at[i]], rows[b], gsem[b])
            pltpu.async_copy(w.at[i0 + i], wbufs[b], gsem[b])

        def gather_wait(i, b):
            pltpu.make_async_copy(ebuf.at[srcall.at[i]], rows[b],
                                  gsem[b]).wait()
            pltpu.make_async_copy(w.at[i0 + i], wbufs[b], gsem[b]).wait()

        def scatter_issue(i, b):
            pltpu.async_copy(rows[b], accum.at[dstall.at[i]], ssem[b],
                             add=True)

        def scatter_wait(i, b):
            pltpu.make_async_copy(rows[b], accum.at[dstall.at[i]],
                                  ssem[b]).wait()

        def scale(i, b):
            rb, wvb = rows[b], wbufs[b]

            def body(j2, _):
                for u in range(2):
                    j = j2 * 2 + u
                    wvec = plsc.load_gather(wvb,
                                            [jnp.broadcast_to(j, (16,))])
                    for q in range(NV):
                        sl = pl.ds(q * 16, 16)
                        rb[j, sl] = rb[j, sl] * wvec
                return 0
            lax.fori_loop(0, CHUNK // 2, body, 0)

        pltpu.sync_copy(tbl.at[pl.ds(r0, ROWS_PER_TILE)], sumv)
        pltpu.sync_copy(sumv, ebuf.at[pl.ds(r0, ROWS_PER_TILE)])
        plsc.subcore_barrier()

        def layer_body(_l, _c):
            def zero(b, _):
                pltpu.sync_copy(zbuf, accum.at[pl.ds(r0 + b * ZB, ZB)])
                return 0
            lax.fori_loop(0, ROWS_PER_TILE // ZB, zero, 0)
            plsc.subcore_barrier()

            gather_issue(0, 0)
            gather_issue(1, 1)

            def ring(t, _):
                for slot in range(NBUF):
                    i = NBUF * t + slot
                    nxt = (slot + 2) % NBUF
                    if slot < 2:
                        pl.when(t > 0)(lambda: scatter_wait(i - 2, nxt))
                    else:
                        scatter_wait(i - 2, nxt)
                    gather_issue(i + 2, nxt)
                    gather_wait(i, slot)
                    scale(i, slot)
                    scatter_issue(i, slot)
                return 0
            lax.fori_loop(0, T_RING, ring, 0)

            for i in range(L_TAIL, N_CHUNKS):
                slot = i % NBUF
                if i + 2 < N_CHUNKS:
                    scatter_wait(i - 2, (i + 2) % NBUF)
                    gather_issue(i + 2, (i + 2) % NBUF)
                gather_wait(i, slot)
                scale(i, slot)
                scatter_issue(i, slot)
            for k in range(N_CHUNKS - NBUF, N_CHUNKS):
                scatter_wait(k, k % NBUF)
            plsc.subcore_barrier()

            def wb(b, _):
                rb = r0 + b * WB
                pltpu.sync_copy(accum.at[pl.ds(rb, WB)], tmp)
                pltpu.sync_copy(tmp, ebuf.at[pl.ds(rb, WB)])

                def acc(i, _):
                    for q in range(NV):
                        sl = pl.ds(q * 16, 16)
                        sumv[b * WB + i, sl] = sumv[b * WB + i, sl] + tmp[i, sl]
                    return 0
                lax.fori_loop(0, WB, acc, 0)
                return 0
            lax.fori_loop(0, N_WB, wb, 0)
            plsc.subcore_barrier()
            return 0
        lax.fori_loop(0, N_LAYERS, layer_body, 0)

        inv = jnp.float32(1.0 / (N_LAYERS + 1))

        def finb(b, _):
            rb = r0 + b * WB

            def fin(i, _):
                for q in range(NV):
                    sl = pl.ds(q * 16, 16)
                    tmp[i, sl] = sumv[b * WB + i, sl] * inv
                return 0
            lax.fori_loop(0, WB, fin, 0)
            pltpu.sync_copy(tmp, out.at[pl.ds(rb, WB)])
            return 0
        lax.fori_loop(0, N_WB, finb, 0)

    def core0():
        run(t0, eb0, out0)
        run(t1, eb1, out1)

    def core1():
        run(t2, eb2, out2)
        run(t3, eb3, out3)

    pl.when(c == 0)(core0)
    pl.when(c == 1)(core1)


@jax.jit
def kernel(user_emb, item_emb, edge_src, edge_dst, edge_weight):
    ego = jnp.concatenate([user_emb, item_emb], axis=0)
    ego = jnp.pad(ego, ((0, N_PAD - N_NODES), (0, 0)))
    tables = [ego[:, b * BLK:(b + 1) * BLK] for b in range(N_BLK)]
    src = edge_src.astype(jnp.int32).reshape(N_EDGES // CHUNK, CHUNK)
    dst = edge_dst.astype(jnp.int32).reshape(N_EDGES // CHUNK, CHUNK)
    w = edge_weight.astype(jnp.float32).reshape(N_EDGES // CHUNK, CHUNK)

    mesh = plsc.VectorSubcoreMesh(core_axis_name="c", subcore_axis_name="s")
    f32 = jnp.float32
    i32 = jnp.int32
    blk_t = jax.ShapeDtypeStruct((N_PAD, BLK), f32)
    call = pl.kernel(
        _lightgcn_body,
        out_type=[blk_t] * 8,
        mesh=mesh,
        compiler_params=pltpu.CompilerParams(
            needs_layout_passes=False, use_tc_tiling_on_sc=False),
        scratch_types=[
            pltpu.VMEM_SHARED((N_PAD, BLK), f32),
            pltpu.VMEM((ROWS_PER_TILE, BLK), f32),
            pltpu.VMEM((N_CHUNKS, CHUNK), i32),
            pltpu.VMEM((N_CHUNKS, CHUNK), i32),
            pltpu.VMEM((CHUNK, BLK), f32),
            pltpu.VMEM((CHUNK, BLK), f32),
            pltpu.VMEM((CHUNK, BLK), f32),
            pltpu.VMEM((CHUNK, BLK), f32),
            pltpu.VMEM((CHUNK,), f32),
            pltpu.VMEM((CHUNK,), f32),
            pltpu.VMEM((CHUNK,), f32),
            pltpu.VMEM((CHUNK,), f32),
            pltpu.VMEM((WB, BLK), f32),
            pltpu.VMEM((ZB, BLK), f32),
            pltpu.SemaphoreType.DMA,
            pltpu.SemaphoreType.DMA,
            pltpu.SemaphoreType.DMA,
            pltpu.SemaphoreType.DMA,
            pltpu.SemaphoreType.DMA,
            pltpu.SemaphoreType.DMA,
            pltpu.SemaphoreType.DMA,
            pltpu.SemaphoreType.DMA,
        ],
    )
    outs = call(*tables, src, dst, w)
    mean_emb = jnp.concatenate(outs[:N_BLK], axis=1)
    return (mean_emb[:N_USERS], mean_emb[N_USERS:N_NODES])
